# baseline (speedup 1.0000x reference)
import jax
import jax.numpy as jnp
from jax.experimental import pallas as pl
from jax.experimental.pallas import tpu as pltpu

HIDDEN = 1024
FFN = 2048
E = 8
TOPK = 2
T = 256

FC = 512
NF = FFN // FC
NEG = -1e30


def _router(x, gw):
    logits = jax.lax.dot_general(
        x.astype(jnp.bfloat16), gw.astype(jnp.bfloat16),
        (((1,), (1,)), ((), ())),
        preferred_element_type=jnp.float32)
    m = jnp.max(logits, axis=1, keepdims=True)
    p = jnp.exp(logits - m)
    sm = p / jnp.sum(p, axis=1, keepdims=True)
    v1 = jnp.max(sm, axis=1, keepdims=True)
    sm2 = jnp.where(sm == v1, NEG, sm)
    v2 = jnp.max(sm2, axis=1, keepdims=True)
    s = v1 + v2
    combine = jnp.where(sm >= v2, sm / s, 0.0)
    return logits, combine


def _moe_body(x_ref, gw_ref, g_ref, u_ref, w2_ref,
              out_ref, logits_ref, comb_ref):
    e = pl.program_id(0)
    f = pl.program_id(1)

    @pl.when(jnp.logical_and(e == 0, f == 0))
    def _():
        logits, combine = _router(x_ref[...], gw_ref[...])
        logits_ref[...] = logits
        comb_ref[...] = combine

    x = x_ref[...].astype(jnp.bfloat16)
    g = g_ref[0].astype(jnp.bfloat16)
    u = u_ref[0].astype(jnp.bfloat16)
    w2c = w2_ref[0].astype(jnp.bfloat16)
    hg = jax.lax.dot_general(x, g, (((1,), (1,)), ((), ())),
                             preferred_element_type=jnp.float32)
    hu = jax.lax.dot_general(x, u, (((1,), (1,)), ((), ())),
                             preferred_element_type=jnp.float32)
    act = (hg * jax.lax.logistic(hg) * hu).astype(jnp.bfloat16)
    part = jax.lax.dot_general(act, w2c, (((1,), (1,)), ((), ())),
                               preferred_element_type=jnp.float32)
    eidx = jax.lax.broadcasted_iota(jnp.int32, (T, E), 1)
    scale = jnp.sum(jnp.where(eidx == e, comb_ref[...], 0.0),
                    axis=1, keepdims=True)
    part = part * scale

    @pl.when(jnp.logical_and(e == 0, f == 0))
    def _():
        out_ref[...] = part

    @pl.when(jnp.logical_or(e != 0, f != 0))
    def _():
        out_ref[...] = out_ref[...] + part


def kernel(hidden_states, gate_w, w13, w2):
    b, s, h = hidden_states.shape
    x = hidden_states.reshape(-1, h)

    out, logits = pl.pallas_call(
        _moe_body,
        grid=(E, NF),
        in_specs=[
            pl.BlockSpec((T, HIDDEN), lambda e, f: (0, 0)),
            pl.BlockSpec((E, HIDDEN), lambda e, f: (0, 0)),
            pl.BlockSpec((1, FC, HIDDEN), lambda e, f: (e, f, 0)),
            pl.BlockSpec((1, FC, HIDDEN), lambda e, f: (e, f + NF, 0)),
            pl.BlockSpec((1, HIDDEN, FC), lambda e, f: (e, 0, f)),
        ],
        out_specs=[
            pl.BlockSpec((T, HIDDEN), lambda e, f: (0, 0)),
            pl.BlockSpec((T, E), lambda e, f: (0, 0)),
        ],
        out_shape=[
            jax.ShapeDtypeStruct((T, HIDDEN), jnp.float32),
            jax.ShapeDtypeStruct((T, E), jnp.float32),
        ],
        scratch_shapes=[pltpu.VMEM((T, E), jnp.float32)],
    )(x, gate_w, w13, w13, w2)

    return out.reshape(b, s, h), logits


# speedup vs baseline: 1.0552x; 1.0552x over previous
import jax
import jax.numpy as jnp
from jax.experimental import pallas as pl
from jax.experimental.pallas import tpu as pltpu

HIDDEN = 1024
FFN = 2048
E = 8
T = 256
NEG = -1e30


def _router(x, gw):
    logits = jax.lax.dot_general(
        x.astype(jnp.bfloat16), gw.astype(jnp.bfloat16),
        (((1,), (1,)), ((), ())),
        preferred_element_type=jnp.float32)
    m = jnp.max(logits, axis=1, keepdims=True)
    p = jnp.exp(logits - m)
    sm = p / jnp.sum(p, axis=1, keepdims=True)
    v1 = jnp.max(sm, axis=1, keepdims=True)
    sm2 = jnp.where(sm == v1, NEG, sm)
    v2 = jnp.max(sm2, axis=1, keepdims=True)
    s = v1 + v2
    combine = jnp.where(sm >= v2, sm / s, 0.0)
    return logits, combine


def _moe_body(x_ref, gw_ref, g_ref, u_ref, w2_ref,
              out_ref, logits_ref, comb_ref):
    e = pl.program_id(0)

    @pl.when(e == 0)
    def _():
        logits, combine = _router(x_ref[...], gw_ref[...])
        logits_ref[...] = logits
        comb_ref[...] = combine

    x = x_ref[...].astype(jnp.bfloat16)
    g = g_ref[0].astype(jnp.bfloat16)
    u = u_ref[0].astype(jnp.bfloat16)
    w2c = w2_ref[0].astype(jnp.bfloat16)
    hg = jax.lax.dot_general(x, g, (((1,), (1,)), ((), ())),
                             preferred_element_type=jnp.float32)
    hu = jax.lax.dot_general(x, u, (((1,), (1,)), ((), ())),
                             preferred_element_type=jnp.float32)
    act = (hg * jax.lax.logistic(hg) * hu).astype(jnp.bfloat16)
    part = jax.lax.dot_general(act, w2c, (((1,), (1,)), ((), ())),
                               preferred_element_type=jnp.float32)
    eidx = jax.lax.broadcasted_iota(jnp.int32, (T, E), 1)
    scale = jnp.sum(jnp.where(eidx == e, comb_ref[...], 0.0),
                    axis=1, keepdims=True)
    part = part * scale

    @pl.when(e == 0)
    def _():
        out_ref[...] = part

    @pl.when(e != 0)
    def _():
        out_ref[...] = out_ref[...] + part


def kernel(hidden_states, gate_w, w13, w2):
    b, s, h = hidden_states.shape
    x = hidden_states.reshape(-1, h)

    out, logits = pl.pallas_call(
        _moe_body,
        grid=(E,),
        in_specs=[
            pl.BlockSpec((T, HIDDEN), lambda e: (0, 0)),
            pl.BlockSpec((E, HIDDEN), lambda e: (0, 0)),
            pl.BlockSpec((1, FFN, HIDDEN), lambda e: (e, 0, 0)),
            pl.BlockSpec((1, FFN, HIDDEN), lambda e: (e, 1, 0)),
            pl.BlockSpec((1, HIDDEN, FFN), lambda e: (e, 0, 0)),
        ],
        out_specs=[
            pl.BlockSpec((T, HIDDEN), lambda e: (0, 0)),
            pl.BlockSpec((T, E), lambda e: (0, 0)),
        ],
        out_shape=[
            jax.ShapeDtypeStruct((T, HIDDEN), jnp.float32),
            jax.ShapeDtypeStruct((T, E), jnp.float32),
        ],
        scratch_shapes=[pltpu.VMEM((T, E), jnp.float32)],
    )(x, gate_w, w13, w13, w2)

    return out.reshape(b, s, h), logits


# speedup vs baseline: 1.0686x; 1.0127x over previous
import jax
import jax.numpy as jnp
from jax.experimental import pallas as pl
from jax.experimental.pallas import tpu as pltpu

HIDDEN = 1024
FFN = 2048
E = 8
T = 256
NEG = -1e30


def _router(x, gw):
    logits = jax.lax.dot_general(
        x.astype(jnp.bfloat16), gw.astype(jnp.bfloat16),
        (((1,), (1,)), ((), ())),
        preferred_element_type=jnp.float32)
    m = jnp.max(logits, axis=1, keepdims=True)
    p = jnp.exp(logits - m)
    sm = p / jnp.sum(p, axis=1, keepdims=True)
    v1 = jnp.max(sm, axis=1, keepdims=True)
    sm2 = jnp.where(sm == v1, NEG, sm)
    v2 = jnp.max(sm2, axis=1, keepdims=True)
    s = v1 + v2
    combine = jnp.where(sm >= v2, sm / s, 0.0)
    return logits, combine


def _moe_body(x_ref, gw_ref, w13_hbm, w2_hbm, out_ref, logits_ref,
              w13buf, w2buf, sem13, sem2):
    def copy13(e, slot):
        return pltpu.make_async_copy(
            w13_hbm.at[e], w13buf.at[slot], sem13.at[slot])

    def copy2(e, slot):
        return pltpu.make_async_copy(
            w2_hbm.at[e], w2buf.at[slot], sem2.at[slot])

    copy13(0, 0).start()
    copy2(0, 0).start()
    copy13(1, 1).start()
    copy2(1, 1).start()

    logits, combine = _router(x_ref[...], gw_ref[...])
    logits_ref[...] = logits
    x = x_ref[...].astype(jnp.bfloat16)

    for e in range(E):
        slot = e % 2
        copy13(e, slot).wait()
        copy2(e, slot).wait()
        g = w13buf[slot, :FFN].astype(jnp.bfloat16)
        u = w13buf[slot, FFN:].astype(jnp.bfloat16)
        w2c = w2buf[slot].astype(jnp.bfloat16)
        hg = jax.lax.dot_general(x, g, (((1,), (1,)), ((), ())),
                                 preferred_element_type=jnp.float32)
        hu = jax.lax.dot_general(x, u, (((1,), (1,)), ((), ())),
                                 preferred_element_type=jnp.float32)
        act = (hg * jax.lax.logistic(hg) * hu).astype(jnp.bfloat16)
        part = jax.lax.dot_general(act, w2c, (((1,), (1,)), ((), ())),
                                   preferred_element_type=jnp.float32)
        part = part * combine[:, e:e + 1]
        if e == 0:
            out_ref[...] = part
        else:
            out_ref[...] = out_ref[...] + part
        if e + 2 < E:
            copy13(e + 2, slot).start()
            copy2(e + 2, slot).start()


def kernel(hidden_states, gate_w, w13, w2):
    b, s, h = hidden_states.shape
    x = hidden_states.reshape(-1, h)

    out, logits = pl.pallas_call(
        _moe_body,
        in_specs=[
            pl.BlockSpec(memory_space=pltpu.VMEM),
            pl.BlockSpec(memory_space=pltpu.VMEM),
            pl.BlockSpec(memory_space=pltpu.HBM),
            pl.BlockSpec(memory_space=pltpu.HBM),
        ],
        out_specs=[
            pl.BlockSpec(memory_space=pltpu.VMEM),
            pl.BlockSpec(memory_space=pltpu.VMEM),
        ],
        out_shape=[
            jax.ShapeDtypeStruct((T, HIDDEN), jnp.float32),
            jax.ShapeDtypeStruct((T, E), jnp.float32),
        ],
        scratch_shapes=[
            pltpu.VMEM((2, 2 * FFN, HIDDEN), jnp.float32),
            pltpu.VMEM((2, HIDDEN, FFN), jnp.float32),
            pltpu.SemaphoreType.DMA((2,)),
            pltpu.SemaphoreType.DMA((2,)),
        ],
    )(x, gate_w, w13, w2)

    return out.reshape(b, s, h), logits


# speedup vs baseline: 1.0739x; 1.0050x over previous
import jax
import jax.numpy as jnp
from jax.experimental import pallas as pl
from jax.experimental.pallas import tpu as pltpu

HIDDEN = 1024
FFN = 2048
E = 8
T = 256
NEG = -1e30


def _router(x, gw):
    logits = jax.lax.dot_general(
        x.astype(jnp.bfloat16), gw.astype(jnp.bfloat16),
        (((1,), (1,)), ((), ())),
        preferred_element_type=jnp.float32)
    m = jnp.max(logits, axis=1, keepdims=True)
    p = jnp.exp(logits - m)
    sm = p / jnp.sum(p, axis=1, keepdims=True)
    v1 = jnp.max(sm, axis=1, keepdims=True)
    sm2 = jnp.where(sm == v1, NEG, sm)
    v2 = jnp.max(sm2, axis=1, keepdims=True)
    s = v1 + v2
    combine = jnp.where(sm >= v2, sm / s, 0.0)
    return logits, combine


def _moe_body(x_ref, gw_ref, w13_hbm, w2_hbm, out_ref, logits_ref,
              w13buf, w2buf, sem13, sem2):
    def copy13(e, slot):
        return pltpu.make_async_copy(
            w13_hbm.at[e], w13buf.at[slot], sem13.at[slot])

    def copy2(e, slot):
        return pltpu.make_async_copy(
            w2_hbm.at[e], w2buf.at[slot], sem2.at[slot])

    copy13(0, 0).start()
    copy2(0, 0).start()
    copy13(1, 1).start()
    copy2(1, 1).start()

    logits, combine = _router(x_ref[...], gw_ref[...])
    logits_ref[...] = logits
    x = x_ref[...]

    for e in range(E):
        slot = e % 2
        copy13(e, slot).wait()
        copy2(e, slot).wait()
        g = w13buf[slot, :FFN]
        u = w13buf[slot, FFN:]
        w2c = w2buf[slot]
        hg = jax.lax.dot_general(x, g, (((1,), (1,)), ((), ())),
                                 preferred_element_type=jnp.float32)
        hu = jax.lax.dot_general(x, u, (((1,), (1,)), ((), ())),
                                 preferred_element_type=jnp.float32)
        act = hg * jax.lax.logistic(hg) * hu
        part = jax.lax.dot_general(act, w2c, (((1,), (1,)), ((), ())),
                                   preferred_element_type=jnp.float32)
        part = part * combine[:, e:e + 1]
        if e == 0:
            out_ref[...] = part
        else:
            out_ref[...] = out_ref[...] + part
        if e + 2 < E:
            copy13(e + 2, slot).start()
            copy2(e + 2, slot).start()


def kernel(hidden_states, gate_w, w13, w2):
    b, s, h = hidden_states.shape
    x = hidden_states.reshape(-1, h)

    out, logits = pl.pallas_call(
        _moe_body,
        in_specs=[
            pl.BlockSpec(memory_space=pltpu.VMEM),
            pl.BlockSpec(memory_space=pltpu.VMEM),
            pl.BlockSpec(memory_space=pltpu.HBM),
            pl.BlockSpec(memory_space=pltpu.HBM),
        ],
        out_specs=[
            pl.BlockSpec(memory_space=pltpu.VMEM),
            pl.BlockSpec(memory_space=pltpu.VMEM),
        ],
        out_shape=[
            jax.ShapeDtypeStruct((T, HIDDEN), jnp.float32),
            jax.ShapeDtypeStruct((T, E), jnp.float32),
        ],
        scratch_shapes=[
            pltpu.VMEM((2, 2 * FFN, HIDDEN), jnp.float32),
            pltpu.VMEM((2, HIDDEN, FFN), jnp.float32),
            pltpu.SemaphoreType.DMA((2,)),
            pltpu.SemaphoreType.DMA((2,)),
        ],
    )(x, gate_w, w13, w2)

    return out.reshape(b, s, h), logits


# speedup vs baseline: 1.1212x; 1.0440x over previous
import jax
import jax.numpy as jnp
from jax.experimental import pallas as pl
from jax.experimental.pallas import tpu as pltpu

HIDDEN = 1024
FFN = 2048
E = 8
T = 256
NEG = -1e30


def _router(x, gw):
    logits = jax.lax.dot_general(
        x.astype(jnp.bfloat16), gw.astype(jnp.bfloat16),
        (((1,), (1,)), ((), ())),
        preferred_element_type=jnp.float32)
    m = jnp.max(logits, axis=1, keepdims=True)
    p = jnp.exp(logits - m)
    sm = p / jnp.sum(p, axis=1, keepdims=True)
    v1 = jnp.max(sm, axis=1, keepdims=True)
    sm2 = jnp.where(sm == v1, NEG, sm)
    v2 = jnp.max(sm2, axis=1, keepdims=True)
    s = v1 + v2
    combine = jnp.where(sm >= v2, sm / s, 0.0)
    return logits, combine


def _moe_body(x_ref, gw_ref, w13_hbm, w2_hbm, out_ref, logits_ref,
              w13buf, w2buf, sem13, sem2):
    def copy13(e, slot):
        return pltpu.make_async_copy(
            w13_hbm.at[e], w13buf.at[slot], sem13.at[slot])

    def copy2(e, slot):
        return pltpu.make_async_copy(
            w2_hbm.at[e], w2buf.at[slot], sem2.at[slot])

    copy13(0, 0).start()
    copy2(0, 0).start()
    copy13(1, 1).start()
    copy2(1, 1).start()

    logits, combine = _router(x_ref[...], gw_ref[...])
    logits_ref[...] = logits
    x = x_ref[...]

    for e in range(E):
        slot = e % 2
        copy13(e, slot).wait()
        g = w13buf[slot, :FFN]
        u = w13buf[slot, FFN:]
        hg = jax.lax.dot_general(x, g, (((1,), (1,)), ((), ())),
                                 preferred_element_type=jnp.float32)
        hu = jax.lax.dot_general(x, u, (((1,), (1,)), ((), ())),
                                 preferred_element_type=jnp.float32)
        act = hg * jax.lax.logistic(hg) * hu
        copy2(e, slot).wait()
        w2c = w2buf[slot]
        part = jax.lax.dot_general(act, w2c, (((1,), (1,)), ((), ())),
                                   preferred_element_type=jnp.float32)
        part = part * combine[:, e:e + 1]
        if e == 0:
            out_ref[...] = part
        else:
            out_ref[...] = out_ref[...] + part
        if e + 2 < E:
            copy13(e + 2, slot).start()
            copy2(e + 2, slot).start()


def kernel(hidden_states, gate_w, w13, w2):
    b, s, h = hidden_states.shape
    x = hidden_states.reshape(-1, h)

    out, logits = pl.pallas_call(
        _moe_body,
        in_specs=[
            pl.BlockSpec(memory_space=pltpu.VMEM),
            pl.BlockSpec(memory_space=pltpu.VMEM),
            pl.BlockSpec(memory_space=pltpu.HBM),
            pl.BlockSpec(memory_space=pltpu.HBM),
        ],
        out_specs=[
            pl.BlockSpec(memory_space=pltpu.VMEM),
            pl.BlockSpec(memory_space=pltpu.VMEM),
        ],
        out_shape=[
            jax.ShapeDtypeStruct((T, HIDDEN), jnp.float32),
            jax.ShapeDtypeStruct((T, E), jnp.float32),
        ],
        scratch_shapes=[
            pltpu.VMEM((2, 2 * FFN, HIDDEN), jnp.float32),
            pltpu.VMEM((2, HIDDEN, FFN), jnp.float32),
            pltpu.SemaphoreType.DMA((2,)),
            pltpu.SemaphoreType.DMA((2,)),
        ],
    )(x, gate_w, w13, w2)

    return out.reshape(b, s, h), logits
